# Initial kernel scaffold; baseline (speedup 1.0000x reference)
import jax
import jax.numpy as jnp
from jax.experimental import pallas as pl


def kernel(x, edge_index, batch, W_enc, b_enc, Wg0, bg0, Wg1, bg1, Wg2, bg2, Wg3, bg3, A1, a1, A2, a2, V1, v1, V2, v2):
    raise NotImplementedError("write your pallas kernel here")


# speedup vs baseline: 9.0087x; 9.0087x over previous
import functools

import jax
import jax.numpy as jnp
from jax import lax
from jax.experimental import pallas as pl
from jax.experimental.pallas import tpu as pltpu
from jax.experimental.pallas import tpu_sc as plsc

N = 10000
NPAD = 10240
E = 160000
EPAD = 163840
F = 256
H = 512
G = 64
NC = 2
NS = 16
FC = 4
CH = 128
EB = 128
RPS = NPAD // NS
EPS = EPAD // NS
BPS = EPS // EB
BN = 256
NB = NPAD // BN

@functools.cache
def _mesh():
    return plsc.VectorSubcoreMesh(
        core_axis_name="c", subcore_axis_name="s", num_cores=NC, num_subcores=NS)


def _deg_body(dp_hbm, zeros_hbm, ones_hbm, out_hbm, dpv, ones_v, acc):
    c = lax.axis_index("c")
    s = lax.axis_index("s")
    pltpu.sync_copy(zeros_hbm.at[pl.ds(s * RPS, RPS)], acc.at[pl.ds(s * RPS, RPS)])
    pltpu.sync_copy(ones_hbm, ones_v)
    pltpu.sync_copy(dp_hbm.at[c * NS + s], dpv)
    plsc.subcore_barrier()

    @pl.loop(0, BPS // NC)
    def _(j):
        pltpu.sync_copy(ones_v, acc.at[dpv.at[j]], add=True)

    plsc.subcore_barrier()
    pltpu.sync_copy(acc.at[pl.ds(s * RPS, RPS)],
                    out_hbm.at[pl.ds(c * NPAD + s * RPS, RPS)])


def _sc_deg(dp_deg, zeros128, ones128):
    return pl.kernel(
        _deg_body,
        out_type=jax.ShapeDtypeStruct((NC * NPAD, CH), jnp.float32),
        mesh=_mesh(),
        scratch_types=[
            pltpu.VMEM((BPS // NC, EB), jnp.int32),
            pltpu.VMEM((EB, CH), jnp.float32),
            pltpu.VMEM_SHARED((NPAD, CH), jnp.float32),
        ],
    )(dp_deg, zeros128, ones128)


def _prop_body(y_hbm, sp4_hbm, dp_hbm, out_hbm, spv, dpv, buf, acc, sem):
    c = lax.axis_index("c")
    s = lax.axis_index("s")
    pltpu.sync_copy(dp_hbm.at[s], dpv)
    for hc in range(FC // NC):
        fc = c * (FC // NC) + hc
        base = fc * NPAD
        pltpu.sync_copy(y_hbm.at[pl.ds(base + s * RPS, RPS)],
                        acc.at[pl.ds(s * RPS, RPS)])
        pltpu.sync_copy(sp4_hbm.at[fc * NS + s], spv)
        plsc.subcore_barrier()

        @pl.loop(0, BPS)
        def _(j):
            pltpu.async_copy(y_hbm.at[spv.at[j]], buf, sem).wait()
            pltpu.sync_copy(buf, acc.at[dpv.at[j]], add=True)

        plsc.subcore_barrier()
        pltpu.sync_copy(acc.at[pl.ds(s * RPS, RPS)],
                        out_hbm.at[pl.ds(base + s * RPS, RPS)])


def _sc_prop(yflat, sp4, dp3):
    return pl.kernel(
        _prop_body,
        out_type=jax.ShapeDtypeStruct((FC * NPAD, CH), jnp.float32),
        mesh=_mesh(),
        scratch_types=[
            pltpu.VMEM((BPS, EB), jnp.int32),
            pltpu.VMEM((BPS, EB), jnp.int32),
            pltpu.VMEM((EB, CH), jnp.float32),
            pltpu.VMEM_SHARED((NPAD, CH), jnp.float32),
            pltpu.SemaphoreType.DMA,
        ],
    )(yflat, sp4, dp3)


def _dinv_of(deg_ref):
    d = deg_ref[0] + deg_ref[1]
    return lax.rsqrt(d[:, 0:1] + 1.0)


def _enc_body(x_ref, W_ref, b_ref, Wg_ref, deg_ref, y_ref):
    dinv = _dinv_of(deg_ref)
    h = jnp.dot(x_ref[...], W_ref[...], preferred_element_type=jnp.float32) + b_ref[0]
    y = jnp.dot(h, Wg_ref[...], preferred_element_type=jnp.float32) * dinv
    for f in range(FC):
        y_ref[f] = y[:, f * CH:(f + 1) * CH]


def _tc_enc(xpad, W_enc, b_enc, Wg0, degs):
    return pl.pallas_call(
        _enc_body,
        grid=(NB,),
        in_specs=[
            pl.BlockSpec((BN, F), lambda i: (i, 0)),
            pl.BlockSpec((F, H), lambda i: (0, 0)),
            pl.BlockSpec((1, H), lambda i: (0, 0)),
            pl.BlockSpec((H, H), lambda i: (0, 0)),
            pl.BlockSpec((NC, BN, CH), lambda i: (0, i, 0)),
        ],
        out_specs=pl.BlockSpec((FC, BN, CH), lambda i: (0, i, 0)),
        out_shape=jax.ShapeDtypeStruct((FC, NPAD, CH), jnp.float32),
    )(xpad, W_enc, b_enc, Wg0, degs)


def _layer_body(prop_ref, deg_ref, b_ref, Wg_ref, y_ref):
    dinv = _dinv_of(deg_ref)
    hcat = jnp.concatenate([prop_ref[f] for f in range(FC)], axis=1)
    h = jnp.maximum(hcat * dinv + b_ref[0], 0.0)
    y = jnp.dot(h, Wg_ref[...], preferred_element_type=jnp.float32) * dinv
    for f in range(FC):
        y_ref[f] = y[:, f * CH:(f + 1) * CH]


def _tc_layer(prop, degs, b_prev, Wg):
    return pl.pallas_call(
        _layer_body,
        grid=(NB,),
        in_specs=[
            pl.BlockSpec((FC, BN, CH), lambda i: (0, i, 0)),
            pl.BlockSpec((NC, BN, CH), lambda i: (0, i, 0)),
            pl.BlockSpec((1, H), lambda i: (0, 0)),
            pl.BlockSpec((H, H), lambda i: (0, 0)),
        ],
        out_specs=pl.BlockSpec((FC, BN, CH), lambda i: (0, i, 0)),
        out_shape=jax.ShapeDtypeStruct((FC, NPAD, CH), jnp.float32),
    )(prop, degs, b_prev, Wg)


def _head_body(prop_ref, deg_ref, b_ref, A1_ref, a1_ref, A2_ref, a2_ref,
               V1_ref, v1_ref, V2_ref, v2_ref, batch_ref,
               adv_ref, val_ref, am_ref, sums_h, sums_a, cnt):
    i = pl.program_id(0)
    dinv = _dinv_of(deg_ref)
    hcat = jnp.concatenate([prop_ref[f] for f in range(FC)], axis=1)
    h = jnp.maximum(hcat * dinv + b_ref[0], 0.0)
    ap = jnp.maximum(
        jnp.dot(h, A1_ref[...], preferred_element_type=jnp.float32) + a1_ref[0], 0.0)
    adv = jnp.dot(ap, A2_ref[...], preferred_element_type=jnp.float32) + a2_ref[0]
    adv_ref[...] = adv
    b = batch_ref[0, 0]
    oh = (b[:, None] == lax.broadcasted_iota(jnp.int32, (1, G), 1)
          ).astype(jnp.float32)
    ph = lax.dot_general(oh, h, (((0,), (0,)), ((), ())),
                         preferred_element_type=jnp.float32)
    pa = lax.dot_general(oh, adv, (((0,), (0,)), ((), ())),
                         preferred_element_type=jnp.float32)
    pc = jnp.sum(oh, axis=0)[:, None]

    @pl.when(i == 0)
    def _():
        sums_h[...] = ph
        sums_a[...] = pa
        cnt[...] = pc

    @pl.when(i > 0)
    def _():
        sums_h[...] += ph
        sums_a[...] += pa
        cnt[...] += pc

    @pl.when(i == NB - 1)
    def _():
        c = jnp.maximum(cnt[...], 1.0)
        vx = sums_h[...] / c
        v = jnp.dot(
            jnp.maximum(
                jnp.dot(vx, V1_ref[...], preferred_element_type=jnp.float32)
                + v1_ref[0], 0.0),
            V2_ref[...], preferred_element_type=jnp.float32) + v2_ref[0]
        val_ref[...] = v
        am_ref[...] = sums_a[...] / c


def _tc_head(prop, degs, b3, A1, a1, A2, a2, V1, v1, V2, v2, batch3):
    return pl.pallas_call(
        _head_body,
        grid=(NB,),
        in_specs=[
            pl.BlockSpec((FC, BN, CH), lambda i: (0, i, 0)),
            pl.BlockSpec((NC, BN, CH), lambda i: (0, i, 0)),
            pl.BlockSpec((1, H), lambda i: (0, 0)),
            pl.BlockSpec((H, H), lambda i: (0, 0)),
            pl.BlockSpec((1, H), lambda i: (0, 0)),
            pl.BlockSpec((H, 1), lambda i: (0, 0)),
            pl.BlockSpec((1, 1), lambda i: (0, 0)),
            pl.BlockSpec((H, H), lambda i: (0, 0)),
            pl.BlockSpec((1, H), lambda i: (0, 0)),
            pl.BlockSpec((H, 1), lambda i: (0, 0)),
            pl.BlockSpec((1, 1), lambda i: (0, 0)),
            pl.BlockSpec((1, 1, BN), lambda i: (i, 0, 0)),
        ],
        out_specs=[
            pl.BlockSpec((BN, 1), lambda i: (i, 0)),
            pl.BlockSpec((G, 1), lambda i: (0, 0)),
            pl.BlockSpec((G, 1), lambda i: (0, 0)),
        ],
        out_shape=[
            jax.ShapeDtypeStruct((NPAD, 1), jnp.float32),
            jax.ShapeDtypeStruct((G, 1), jnp.float32),
            jax.ShapeDtypeStruct((G, 1), jnp.float32),
        ],
        scratch_shapes=[
            pltpu.VMEM((G, H), jnp.float32),
            pltpu.VMEM((G, 1), jnp.float32),
            pltpu.VMEM((G, 1), jnp.float32),
        ],
    )(prop, degs, b3, A1, a1, A2, a2, V1, v1, V2, v2, batch3)


def _out_body(adv_ref, batch_ref, val_ref, am_ref, out_ref):
    b = batch_ref[0, 0]
    oh = (b[:, None] == lax.broadcasted_iota(jnp.int32, (1, G), 1)
          ).astype(jnp.float32)
    delta = val_ref[...] - am_ref[...]
    out_ref[...] = adv_ref[...] + jnp.dot(
        oh, delta, preferred_element_type=jnp.float32)


def _tc_out(adv, batch3, val, am):
    return pl.pallas_call(
        _out_body,
        grid=(NB,),
        in_specs=[
            pl.BlockSpec((BN, 1), lambda i: (i, 0)),
            pl.BlockSpec((1, 1, BN), lambda i: (i, 0, 0)),
            pl.BlockSpec((G, 1), lambda i: (0, 0)),
            pl.BlockSpec((G, 1), lambda i: (0, 0)),
        ],
        out_specs=pl.BlockSpec((BN, 1), lambda i: (i, 0)),
        out_shape=jax.ShapeDtypeStruct((NPAD, 1), jnp.float32),
    )(adv, batch3, val, am)


def kernel(x, edge_index, batch, W_enc, b_enc, Wg0, bg0, Wg1, bg1, Wg2, bg2,
           Wg3, bg3, A1, a1, A2, a2, V1, v1, V2, v2):
    src, dst = edge_index[0], edge_index[1]
    pad_idx = (N + jnp.arange(EPAD - E, dtype=jnp.int32) % (NPAD - N))
    sp = jnp.concatenate([src, pad_idx])
    dp = jnp.concatenate([dst, pad_idx])
    sp4 = (sp[None, :]
           + (jnp.arange(FC, dtype=jnp.int32) * NPAD)[:, None]
           ).reshape(FC * NS, BPS, EB)
    dp3 = dp.reshape(NS, BPS, EB)
    xpad = jnp.concatenate([x, jnp.zeros((NPAD - N, F), x.dtype)])
    batch3 = jnp.concatenate(
        [batch, jnp.full((NPAD - N,), G, jnp.int32)]).reshape(NB, 1, BN)
    zeros128 = jnp.zeros((NPAD, CH), jnp.float32)
    ones128 = jnp.ones((EB, CH), jnp.float32)

    dp_deg = dp.reshape(NC * NS, BPS // NC, EB)
    degs = _sc_deg(dp_deg, zeros128, ones128).reshape(NC, NPAD, CH)
    y = _tc_enc(xpad, W_enc, b_enc.reshape(1, H), Wg0, degs)
    for Wg, b_prev in ((Wg1, bg0), (Wg2, bg1), (Wg3, bg2)):
        prop = _sc_prop(y.reshape(FC * NPAD, CH), sp4, dp3).reshape(FC, NPAD, CH)
        y = _tc_layer(prop, degs, b_prev.reshape(1, H), Wg)
    prop = _sc_prop(y.reshape(FC * NPAD, CH), sp4, dp3).reshape(FC, NPAD, CH)
    adv, val, am = _tc_head(prop, degs, bg3.reshape(1, H), A1,
                            a1.reshape(1, H), A2, a2.reshape(1, 1),
                            V1, v1.reshape(1, H), V2, v2.reshape(1, 1), batch3)
    out = _tc_out(adv, batch3, val, am)
    return out[:N]


# speedup vs baseline: 12.5733x; 1.3957x over previous
import functools

import jax
import jax.numpy as jnp
from jax import lax
from jax.experimental import pallas as pl
from jax.experimental.pallas import tpu as pltpu
from jax.experimental.pallas import tpu_sc as plsc

N = 10000
NPAD = 10240
E = 160000
EPAD = 163840
F = 256
H = 512
G = 64
NC = 2
NS = 16
FC = 4
CH = 128
EB = 128
RPS = NPAD // NS
EPS = EPAD // NS
BPS = EPS // EB
BN = 256
NB = NPAD // BN

@functools.cache
def _mesh():
    return plsc.VectorSubcoreMesh(
        core_axis_name="c", subcore_axis_name="s", num_cores=NC, num_subcores=NS)


def _deg_body(dp_hbm, zeros_hbm, ones_hbm, out_hbm, dpv, ones_v, acc):
    c = lax.axis_index("c")
    s = lax.axis_index("s")
    pltpu.sync_copy(zeros_hbm.at[pl.ds(s * RPS, RPS)], acc.at[pl.ds(s * RPS, RPS)])
    pltpu.sync_copy(ones_hbm, ones_v)
    pltpu.sync_copy(dp_hbm.at[c * NS + s], dpv)
    plsc.subcore_barrier()

    @pl.loop(0, BPS // NC)
    def _(j):
        pltpu.sync_copy(ones_v, acc.at[dpv.at[j]], add=True)

    plsc.subcore_barrier()
    pltpu.sync_copy(acc.at[pl.ds(s * RPS, RPS)],
                    out_hbm.at[pl.ds(c * NPAD + s * RPS, RPS)])


def _sc_deg(dp_deg, zeros128, ones128):
    return pl.kernel(
        _deg_body,
        out_type=jax.ShapeDtypeStruct((NC * NPAD, CH), jnp.float32),
        mesh=_mesh(),
        scratch_types=[
            pltpu.VMEM((BPS // NC, EB), jnp.int32),
            pltpu.VMEM((EB, CH), jnp.float32),
            pltpu.VMEM_SHARED((NPAD, CH), jnp.float32),
        ],
    )(dp_deg, zeros128, ones128)


SEG = 4
SROWS = BPS // SEG


def _prop_body(y_hbm, sp_hbm, dp_hbm, out_hbm, spv, dpv, buf, acc, sem0, sem1):
    c = lax.axis_index("c")
    s = lax.axis_index("s")
    for hc in range(FC // NC):
        fc = c * (FC // NC) + hc
        base = fc * NPAD
        pltpu.sync_copy(y_hbm.at[pl.ds(base + s * RPS, RPS)],
                        acc.at[pl.ds(s * RPS, RPS)])
        plsc.subcore_barrier()
        for seg in range(SEG):
            pltpu.sync_copy(sp_hbm.at[(fc * NS + s) * SEG + seg], spv)
            pltpu.sync_copy(dp_hbm.at[s * SEG + seg], dpv)
            pltpu.async_copy(y_hbm.at[spv.at[0]], buf.at[0], sem0)

            @pl.loop(0, SROWS // 2)
            def _(t):
                j = 2 * t
                pltpu.async_copy(y_hbm.at[spv.at[j + 1]], buf.at[1], sem1)
                pltpu.make_async_copy(y_hbm.at[spv.at[j]], buf.at[0], sem0).wait()
                pltpu.sync_copy(buf.at[0], acc.at[dpv.at[j]], add=True)

                @pl.when(j + 2 < SROWS)
                def _():
                    pltpu.async_copy(y_hbm.at[spv.at[j + 2]], buf.at[0], sem0)

                pltpu.make_async_copy(y_hbm.at[spv.at[j + 1]], buf.at[1], sem1).wait()
                pltpu.sync_copy(buf.at[1], acc.at[dpv.at[j + 1]], add=True)

        plsc.subcore_barrier()
        pltpu.sync_copy(acc.at[pl.ds(s * RPS, RPS)],
                        out_hbm.at[pl.ds(base + s * RPS, RPS)])


def _sc_prop(yflat, spseg, dpseg):
    return pl.kernel(
        _prop_body,
        out_type=jax.ShapeDtypeStruct((FC * NPAD, CH), jnp.float32),
        mesh=_mesh(),
        scratch_types=[
            pltpu.VMEM((SROWS, EB), jnp.int32),
            pltpu.VMEM((SROWS, EB), jnp.int32),
            pltpu.VMEM((2, EB, CH), jnp.float32),
            pltpu.VMEM_SHARED((NPAD, CH), jnp.float32),
            pltpu.SemaphoreType.DMA,
            pltpu.SemaphoreType.DMA,
        ],
    )(yflat, spseg, dpseg)


def _dinv_of(deg_ref):
    d = deg_ref[0] + deg_ref[1]
    return lax.rsqrt(d[:, 0:1] + 1.0)


def _enc_body(x_ref, W_ref, b_ref, Wg_ref, deg_ref, y_ref):
    dinv = _dinv_of(deg_ref)
    h = jnp.dot(x_ref[...], W_ref[...], preferred_element_type=jnp.float32) + b_ref[0]
    y = jnp.dot(h, Wg_ref[...], preferred_element_type=jnp.float32) * dinv
    for f in range(FC):
        y_ref[f] = y[:, f * CH:(f + 1) * CH]


def _tc_enc(xpad, W_enc, b_enc, Wg0, degs):
    return pl.pallas_call(
        _enc_body,
        grid=(NB,),
        in_specs=[
            pl.BlockSpec((BN, F), lambda i: (i, 0)),
            pl.BlockSpec((F, H), lambda i: (0, 0)),
            pl.BlockSpec((1, H), lambda i: (0, 0)),
            pl.BlockSpec((H, H), lambda i: (0, 0)),
            pl.BlockSpec((NC, BN, CH), lambda i: (0, i, 0)),
        ],
        out_specs=pl.BlockSpec((FC, BN, CH), lambda i: (0, i, 0)),
        out_shape=jax.ShapeDtypeStruct((FC, NPAD, CH), jnp.float32),
    )(xpad, W_enc, b_enc, Wg0, degs)


def _layer_body(prop_ref, deg_ref, b_ref, Wg_ref, y_ref):
    dinv = _dinv_of(deg_ref)
    hcat = jnp.concatenate([prop_ref[f] for f in range(FC)], axis=1)
    h = jnp.maximum(hcat * dinv + b_ref[0], 0.0)
    y = jnp.dot(h, Wg_ref[...], preferred_element_type=jnp.float32) * dinv
    for f in range(FC):
        y_ref[f] = y[:, f * CH:(f + 1) * CH]


def _tc_layer(prop, degs, b_prev, Wg):
    return pl.pallas_call(
        _layer_body,
        grid=(NB,),
        in_specs=[
            pl.BlockSpec((FC, BN, CH), lambda i: (0, i, 0)),
            pl.BlockSpec((NC, BN, CH), lambda i: (0, i, 0)),
            pl.BlockSpec((1, H), lambda i: (0, 0)),
            pl.BlockSpec((H, H), lambda i: (0, 0)),
        ],
        out_specs=pl.BlockSpec((FC, BN, CH), lambda i: (0, i, 0)),
        out_shape=jax.ShapeDtypeStruct((FC, NPAD, CH), jnp.float32),
    )(prop, degs, b_prev, Wg)


def _head_body(prop_ref, deg_ref, b_ref, A1_ref, a1_ref, A2_ref, a2_ref,
               V1_ref, v1_ref, V2_ref, v2_ref, batch_ref,
               adv_ref, val_ref, am_ref, sums_h, sums_a, cnt):
    i = pl.program_id(0)
    dinv = _dinv_of(deg_ref)
    hcat = jnp.concatenate([prop_ref[f] for f in range(FC)], axis=1)
    h = jnp.maximum(hcat * dinv + b_ref[0], 0.0)
    ap = jnp.maximum(
        jnp.dot(h, A1_ref[...], preferred_element_type=jnp.float32) + a1_ref[0], 0.0)
    adv = jnp.dot(ap, A2_ref[...], preferred_element_type=jnp.float32) + a2_ref[0]
    adv_ref[...] = adv
    b = batch_ref[0, 0]
    oh = (b[:, None] == lax.broadcasted_iota(jnp.int32, (1, G), 1)
          ).astype(jnp.float32)
    ph = lax.dot_general(oh, h, (((0,), (0,)), ((), ())),
                         preferred_element_type=jnp.float32)
    pa = lax.dot_general(oh, adv, (((0,), (0,)), ((), ())),
                         preferred_element_type=jnp.float32)
    pc = jnp.sum(oh, axis=0)[:, None]

    @pl.when(i == 0)
    def _():
        sums_h[...] = ph
        sums_a[...] = pa
        cnt[...] = pc

    @pl.when(i > 0)
    def _():
        sums_h[...] += ph
        sums_a[...] += pa
        cnt[...] += pc

    @pl.when(i == NB - 1)
    def _():
        c = jnp.maximum(cnt[...], 1.0)
        vx = sums_h[...] / c
        v = jnp.dot(
            jnp.maximum(
                jnp.dot(vx, V1_ref[...], preferred_element_type=jnp.float32)
                + v1_ref[0], 0.0),
            V2_ref[...], preferred_element_type=jnp.float32) + v2_ref[0]
        val_ref[...] = v
        am_ref[...] = sums_a[...] / c


def _tc_head(prop, degs, b3, A1, a1, A2, a2, V1, v1, V2, v2, batch3):
    return pl.pallas_call(
        _head_body,
        grid=(NB,),
        in_specs=[
            pl.BlockSpec((FC, BN, CH), lambda i: (0, i, 0)),
            pl.BlockSpec((NC, BN, CH), lambda i: (0, i, 0)),
            pl.BlockSpec((1, H), lambda i: (0, 0)),
            pl.BlockSpec((H, H), lambda i: (0, 0)),
            pl.BlockSpec((1, H), lambda i: (0, 0)),
            pl.BlockSpec((H, 1), lambda i: (0, 0)),
            pl.BlockSpec((1, 1), lambda i: (0, 0)),
            pl.BlockSpec((H, H), lambda i: (0, 0)),
            pl.BlockSpec((1, H), lambda i: (0, 0)),
            pl.BlockSpec((H, 1), lambda i: (0, 0)),
            pl.BlockSpec((1, 1), lambda i: (0, 0)),
            pl.BlockSpec((1, 1, BN), lambda i: (i, 0, 0)),
        ],
        out_specs=[
            pl.BlockSpec((BN, 1), lambda i: (i, 0)),
            pl.BlockSpec((G, 1), lambda i: (0, 0)),
            pl.BlockSpec((G, 1), lambda i: (0, 0)),
        ],
        out_shape=[
            jax.ShapeDtypeStruct((NPAD, 1), jnp.float32),
            jax.ShapeDtypeStruct((G, 1), jnp.float32),
            jax.ShapeDtypeStruct((G, 1), jnp.float32),
        ],
        scratch_shapes=[
            pltpu.VMEM((G, H), jnp.float32),
            pltpu.VMEM((G, 1), jnp.float32),
            pltpu.VMEM((G, 1), jnp.float32),
        ],
    )(prop, degs, b3, A1, a1, A2, a2, V1, v1, V2, v2, batch3)


def _out_body(adv_ref, batch_ref, val_ref, am_ref, out_ref):
    b = batch_ref[0, 0]
    oh = (b[:, None] == lax.broadcasted_iota(jnp.int32, (1, G), 1)
          ).astype(jnp.float32)
    delta = val_ref[...] - am_ref[...]
    out_ref[...] = adv_ref[...] + jnp.dot(
        oh, delta, preferred_element_type=jnp.float32)


def _tc_out(adv, batch3, val, am):
    return pl.pallas_call(
        _out_body,
        grid=(NB,),
        in_specs=[
            pl.BlockSpec((BN, 1), lambda i: (i, 0)),
            pl.BlockSpec((1, 1, BN), lambda i: (i, 0, 0)),
            pl.BlockSpec((G, 1), lambda i: (0, 0)),
            pl.BlockSpec((G, 1), lambda i: (0, 0)),
        ],
        out_specs=pl.BlockSpec((BN, 1), lambda i: (i, 0)),
        out_shape=jax.ShapeDtypeStruct((NPAD, 1), jnp.float32),
    )(adv, batch3, val, am)


def kernel(x, edge_index, batch, W_enc, b_enc, Wg0, bg0, Wg1, bg1, Wg2, bg2,
           Wg3, bg3, A1, a1, A2, a2, V1, v1, V2, v2):
    src, dst = edge_index[0], edge_index[1]
    pad_idx = (N + jnp.arange(EPAD - E, dtype=jnp.int32) % (NPAD - N))
    sp = jnp.concatenate([src, pad_idx])
    dp = jnp.concatenate([dst, pad_idx])
    spseg = (sp[None, :]
             + (jnp.arange(FC, dtype=jnp.int32) * NPAD)[:, None]
             ).reshape(FC * NS * SEG, SROWS, EB)
    dpseg = dp.reshape(NS * SEG, SROWS, EB)
    xpad = jnp.concatenate([x, jnp.zeros((NPAD - N, F), x.dtype)])
    batch3 = jnp.concatenate(
        [batch, jnp.full((NPAD - N,), G, jnp.int32)]).reshape(NB, 1, BN)
    zeros128 = jnp.zeros((NPAD, CH), jnp.float32)
    ones128 = jnp.ones((EB, CH), jnp.float32)

    dp_deg = dp.reshape(NC * NS, BPS // NC, EB)
    degs = _sc_deg(dp_deg, zeros128, ones128).reshape(NC, NPAD, CH)
    y = _tc_enc(xpad, W_enc, b_enc.reshape(1, H), Wg0, degs)
    for Wg, b_prev in ((Wg1, bg0), (Wg2, bg1), (Wg3, bg2)):
        prop = _sc_prop(y.reshape(FC * NPAD, CH), spseg, dpseg).reshape(FC, NPAD, CH)
        y = _tc_layer(prop, degs, b_prev.reshape(1, H), Wg)
    prop = _sc_prop(y.reshape(FC * NPAD, CH), spseg, dpseg).reshape(FC, NPAD, CH)
    adv, val, am = _tc_head(prop, degs, bg3.reshape(1, H), A1,
                            a1.reshape(1, H), A2, a2.reshape(1, 1),
                            V1, v1.reshape(1, H), V2, v2.reshape(1, 1), batch3)
    out = _tc_out(adv, batch3, val, am)
    return out[:N]


# speedup vs baseline: 13.0729x; 1.0397x over previous
import functools

import jax
import jax.numpy as jnp
from jax import lax
from jax.experimental import pallas as pl
from jax.experimental.pallas import tpu as pltpu
from jax.experimental.pallas import tpu_sc as plsc

N = 10000
NPAD = 10240
E = 160000
EPAD = 163840
F = 256
H = 512
G = 64
NC = 2
NS = 16
FC = 4
CH = 128
EB = 128
RPS = NPAD // NS
EPS = EPAD // NS
BPS = EPS // EB
BN = 256
NB = NPAD // BN

@functools.cache
def _mesh():
    return plsc.VectorSubcoreMesh(
        core_axis_name="c", subcore_axis_name="s", num_cores=NC, num_subcores=NS)


def _deg_body(dp_hbm, zeros_hbm, ones_hbm, out_hbm, dpv, ones_v, acc):
    c = lax.axis_index("c")
    s = lax.axis_index("s")
    pltpu.sync_copy(zeros_hbm.at[pl.ds(s * RPS, RPS)], acc.at[pl.ds(s * RPS, RPS)])
    pltpu.sync_copy(ones_hbm, ones_v)
    pltpu.sync_copy(dp_hbm.at[c * NS + s], dpv)
    plsc.subcore_barrier()

    @pl.loop(0, BPS // NC)
    def _(j):
        pltpu.sync_copy(ones_v, acc.at[dpv.at[j]], add=True)

    plsc.subcore_barrier()
    pltpu.sync_copy(acc.at[pl.ds(s * RPS, RPS)],
                    out_hbm.at[pl.ds(c * NPAD + s * RPS, RPS)])


def _sc_deg(dp_deg, zeros128, ones128):
    return pl.kernel(
        _deg_body,
        out_type=jax.ShapeDtypeStruct((NC * NPAD, CH), jnp.float32),
        mesh=_mesh(),
        scratch_types=[
            pltpu.VMEM((BPS // NC, EB), jnp.int32),
            pltpu.VMEM((EB, CH), jnp.float32),
            pltpu.VMEM_SHARED((NPAD, CH), jnp.float32),
        ],
    )(dp_deg, zeros128, ones128)


SEG = 2
SROWS = BPS // SEG


def _prop_body(y_hbm, sp_hbm, dp_hbm, out_hbm, spv, dpv, buf, acc, sem0, sem1):
    c = lax.axis_index("c")
    s = lax.axis_index("s")
    for hc in range(FC // NC):
        fc = c * (FC // NC) + hc
        base = fc * NPAD
        pltpu.sync_copy(y_hbm.at[pl.ds(base + s * RPS, RPS)],
                        acc.at[pl.ds(s * RPS, RPS)])
        plsc.subcore_barrier()
        for seg in range(SEG):
            pltpu.sync_copy(sp_hbm.at[(fc * NS + s) * SEG + seg], spv)
            pltpu.sync_copy(dp_hbm.at[s * SEG + seg], dpv)
            pltpu.async_copy(y_hbm.at[spv.at[0]], buf.at[0], sem0)

            @pl.loop(0, SROWS // 2)
            def _(t):
                j = 2 * t
                pltpu.async_copy(y_hbm.at[spv.at[j + 1]], buf.at[1], sem1)
                pltpu.make_async_copy(y_hbm.at[spv.at[j]], buf.at[0], sem0).wait()
                pltpu.sync_copy(buf.at[0], acc.at[dpv.at[j]], add=True)

                @pl.when(j + 2 < SROWS)
                def _():
                    pltpu.async_copy(y_hbm.at[spv.at[j + 2]], buf.at[0], sem0)

                pltpu.make_async_copy(y_hbm.at[spv.at[j + 1]], buf.at[1], sem1).wait()
                pltpu.sync_copy(buf.at[1], acc.at[dpv.at[j + 1]], add=True)

        plsc.subcore_barrier()
        pltpu.sync_copy(acc.at[pl.ds(s * RPS, RPS)],
                        out_hbm.at[pl.ds(base + s * RPS, RPS)])


def _sc_prop(yflat, spseg, dpseg):
    return pl.kernel(
        _prop_body,
        out_type=jax.ShapeDtypeStruct((FC * NPAD, CH), jnp.float32),
        mesh=_mesh(),
        scratch_types=[
            pltpu.VMEM((SROWS, EB), jnp.int32),
            pltpu.VMEM((SROWS, EB), jnp.int32),
            pltpu.VMEM((2, EB, CH), jnp.float32),
            pltpu.VMEM_SHARED((NPAD, CH), jnp.float32),
            pltpu.SemaphoreType.DMA,
            pltpu.SemaphoreType.DMA,
        ],
    )(yflat, spseg, dpseg)


def _dinv_of(deg_ref):
    d = deg_ref[0] + deg_ref[1]
    return lax.rsqrt(d[:, 0:1] + 1.0)


def _enc_body(x_ref, W_ref, b_ref, Wg_ref, deg_ref, y_ref):
    dinv = _dinv_of(deg_ref)
    h = jnp.dot(x_ref[...], W_ref[...], preferred_element_type=jnp.float32) + b_ref[0]
    y = jnp.dot(h, Wg_ref[...], preferred_element_type=jnp.float32) * dinv
    for f in range(FC):
        y_ref[f] = y[:, f * CH:(f + 1) * CH]


def _tc_enc(xpad, W_enc, b_enc, Wg0, degs):
    return pl.pallas_call(
        _enc_body,
        grid=(NB,),
        in_specs=[
            pl.BlockSpec((BN, F), lambda i: (i, 0)),
            pl.BlockSpec((F, H), lambda i: (0, 0)),
            pl.BlockSpec((1, H), lambda i: (0, 0)),
            pl.BlockSpec((H, H), lambda i: (0, 0)),
            pl.BlockSpec((NC, BN, CH), lambda i: (0, i, 0)),
        ],
        out_specs=pl.BlockSpec((FC, BN, CH), lambda i: (0, i, 0)),
        out_shape=jax.ShapeDtypeStruct((FC, NPAD, CH), jnp.float32),
    )(xpad, W_enc, b_enc, Wg0, degs)


def _layer_body(prop_ref, deg_ref, b_ref, Wg_ref, y_ref):
    dinv = _dinv_of(deg_ref)
    hcat = jnp.concatenate([prop_ref[f] for f in range(FC)], axis=1)
    h = jnp.maximum(hcat * dinv + b_ref[0], 0.0)
    y = jnp.dot(h, Wg_ref[...], preferred_element_type=jnp.float32) * dinv
    for f in range(FC):
        y_ref[f] = y[:, f * CH:(f + 1) * CH]


def _tc_layer(prop, degs, b_prev, Wg):
    return pl.pallas_call(
        _layer_body,
        grid=(NB,),
        in_specs=[
            pl.BlockSpec((FC, BN, CH), lambda i: (0, i, 0)),
            pl.BlockSpec((NC, BN, CH), lambda i: (0, i, 0)),
            pl.BlockSpec((1, H), lambda i: (0, 0)),
            pl.BlockSpec((H, H), lambda i: (0, 0)),
        ],
        out_specs=pl.BlockSpec((FC, BN, CH), lambda i: (0, i, 0)),
        out_shape=jax.ShapeDtypeStruct((FC, NPAD, CH), jnp.float32),
    )(prop, degs, b_prev, Wg)


def _head_body(prop_ref, deg_ref, b_ref, A1_ref, a1_ref, A2_ref, a2_ref,
               V1_ref, v1_ref, V2_ref, v2_ref, batch_ref,
               adv_ref, val_ref, am_ref, sums_h, sums_a, cnt):
    i = pl.program_id(0)
    dinv = _dinv_of(deg_ref)
    hcat = jnp.concatenate([prop_ref[f] for f in range(FC)], axis=1)
    h = jnp.maximum(hcat * dinv + b_ref[0], 0.0)
    ap = jnp.maximum(
        jnp.dot(h, A1_ref[...], preferred_element_type=jnp.float32) + a1_ref[0], 0.0)
    adv = jnp.dot(ap, A2_ref[...], preferred_element_type=jnp.float32) + a2_ref[0]
    adv_ref[...] = adv
    b = batch_ref[0, 0]
    oh = (b[:, None] == lax.broadcasted_iota(jnp.int32, (1, G), 1)
          ).astype(jnp.float32)
    ph = lax.dot_general(oh, h, (((0,), (0,)), ((), ())),
                         preferred_element_type=jnp.float32)
    pa = lax.dot_general(oh, adv, (((0,), (0,)), ((), ())),
                         preferred_element_type=jnp.float32)
    pc = jnp.sum(oh, axis=0)[:, None]

    @pl.when(i == 0)
    def _():
        sums_h[...] = ph
        sums_a[...] = pa
        cnt[...] = pc

    @pl.when(i > 0)
    def _():
        sums_h[...] += ph
        sums_a[...] += pa
        cnt[...] += pc

    @pl.when(i == NB - 1)
    def _():
        c = jnp.maximum(cnt[...], 1.0)
        vx = sums_h[...] / c
        v = jnp.dot(
            jnp.maximum(
                jnp.dot(vx, V1_ref[...], preferred_element_type=jnp.float32)
                + v1_ref[0], 0.0),
            V2_ref[...], preferred_element_type=jnp.float32) + v2_ref[0]
        val_ref[...] = v
        am_ref[...] = sums_a[...] / c


def _tc_head(prop, degs, b3, A1, a1, A2, a2, V1, v1, V2, v2, batch3):
    return pl.pallas_call(
        _head_body,
        grid=(NB,),
        in_specs=[
            pl.BlockSpec((FC, BN, CH), lambda i: (0, i, 0)),
            pl.BlockSpec((NC, BN, CH), lambda i: (0, i, 0)),
            pl.BlockSpec((1, H), lambda i: (0, 0)),
            pl.BlockSpec((H, H), lambda i: (0, 0)),
            pl.BlockSpec((1, H), lambda i: (0, 0)),
            pl.BlockSpec((H, 1), lambda i: (0, 0)),
            pl.BlockSpec((1, 1), lambda i: (0, 0)),
            pl.BlockSpec((H, H), lambda i: (0, 0)),
            pl.BlockSpec((1, H), lambda i: (0, 0)),
            pl.BlockSpec((H, 1), lambda i: (0, 0)),
            pl.BlockSpec((1, 1), lambda i: (0, 0)),
            pl.BlockSpec((1, 1, BN), lambda i: (i, 0, 0)),
        ],
        out_specs=[
            pl.BlockSpec((BN, 1), lambda i: (i, 0)),
            pl.BlockSpec((G, 1), lambda i: (0, 0)),
            pl.BlockSpec((G, 1), lambda i: (0, 0)),
        ],
        out_shape=[
            jax.ShapeDtypeStruct((NPAD, 1), jnp.float32),
            jax.ShapeDtypeStruct((G, 1), jnp.float32),
            jax.ShapeDtypeStruct((G, 1), jnp.float32),
        ],
        scratch_shapes=[
            pltpu.VMEM((G, H), jnp.float32),
            pltpu.VMEM((G, 1), jnp.float32),
            pltpu.VMEM((G, 1), jnp.float32),
        ],
    )(prop, degs, b3, A1, a1, A2, a2, V1, v1, V2, v2, batch3)


def _out_body(adv_ref, batch_ref, val_ref, am_ref, out_ref):
    b = batch_ref[0, 0]
    oh = (b[:, None] == lax.broadcasted_iota(jnp.int32, (1, G), 1)
          ).astype(jnp.float32)
    delta = val_ref[...] - am_ref[...]
    out_ref[...] = adv_ref[...] + jnp.dot(
        oh, delta, preferred_element_type=jnp.float32)


def _tc_out(adv, batch3, val, am):
    return pl.pallas_call(
        _out_body,
        grid=(NB,),
        in_specs=[
            pl.BlockSpec((BN, 1), lambda i: (i, 0)),
            pl.BlockSpec((1, 1, BN), lambda i: (i, 0, 0)),
            pl.BlockSpec((G, 1), lambda i: (0, 0)),
            pl.BlockSpec((G, 1), lambda i: (0, 0)),
        ],
        out_specs=pl.BlockSpec((BN, 1), lambda i: (i, 0)),
        out_shape=jax.ShapeDtypeStruct((NPAD, 1), jnp.float32),
    )(adv, batch3, val, am)


def kernel(x, edge_index, batch, W_enc, b_enc, Wg0, bg0, Wg1, bg1, Wg2, bg2,
           Wg3, bg3, A1, a1, A2, a2, V1, v1, V2, v2):
    src, dst = edge_index[0], edge_index[1]
    pad_idx = (N + jnp.arange(EPAD - E, dtype=jnp.int32) % (NPAD - N))
    sp = jnp.concatenate([src, pad_idx])
    dp = jnp.concatenate([dst, pad_idx])
    spseg = (sp[None, :]
             + (jnp.arange(FC, dtype=jnp.int32) * NPAD)[:, None]
             ).reshape(FC * NS * SEG, SROWS, EB)
    dpseg = dp.reshape(NS * SEG, SROWS, EB)
    xpad = jnp.concatenate([x, jnp.zeros((NPAD - N, F), x.dtype)])
    batch3 = jnp.concatenate(
        [batch, jnp.full((NPAD - N,), G, jnp.int32)]).reshape(NB, 1, BN)
    zeros128 = jnp.zeros((NPAD, CH), jnp.float32)
    ones128 = jnp.ones((EB, CH), jnp.float32)

    dp_deg = dp.reshape(NC * NS, BPS // NC, EB)
    degs = _sc_deg(dp_deg, zeros128, ones128).reshape(NC, NPAD, CH)
    y = _tc_enc(xpad, W_enc, b_enc.reshape(1, H), Wg0, degs)
    for Wg, b_prev in ((Wg1, bg0), (Wg2, bg1), (Wg3, bg2)):
        prop = _sc_prop(y.reshape(FC * NPAD, CH), spseg, dpseg).reshape(FC, NPAD, CH)
        y = _tc_layer(prop, degs, b_prev.reshape(1, H), Wg)
    prop = _sc_prop(y.reshape(FC * NPAD, CH), spseg, dpseg).reshape(FC, NPAD, CH)
    adv, val, am = _tc_head(prop, degs, bg3.reshape(1, H), A1,
                            a1.reshape(1, H), A2, a2.reshape(1, 1),
                            V1, v1.reshape(1, H), V2, v2.reshape(1, 1), batch3)
    out = _tc_out(adv, batch3, val, am)
    return out[:N]


# speedup vs baseline: 13.1612x; 1.0068x over previous
import functools

import jax
import jax.numpy as jnp
from jax import lax
from jax.experimental import pallas as pl
from jax.experimental.pallas import tpu as pltpu
from jax.experimental.pallas import tpu_sc as plsc

N = 10000
NPAD = 10240
E = 160000
EPAD = 163840
F = 256
H = 512
G = 64
NC = 2
NS = 16
FC = 4
CH = 128
EB = 128
RPS = NPAD // NS
EPS = EPAD // NS
BPS = EPS // EB
BN = 256
NB = NPAD // BN

@functools.cache
def _mesh():
    return plsc.VectorSubcoreMesh(
        core_axis_name="c", subcore_axis_name="s", num_cores=NC, num_subcores=NS)


def _deg_body(dp_hbm, zeros_hbm, ones_hbm, out_hbm, dpv, ones_v, acc):
    c = lax.axis_index("c")
    s = lax.axis_index("s")
    pltpu.sync_copy(zeros_hbm.at[pl.ds(s * RPS, RPS)], acc.at[pl.ds(s * RPS, RPS)])
    pltpu.sync_copy(ones_hbm, ones_v)
    pltpu.sync_copy(dp_hbm.at[c * NS + s], dpv)
    plsc.subcore_barrier()

    @pl.loop(0, BPS // NC)
    def _(j):
        pltpu.sync_copy(ones_v, acc.at[dpv.at[j]], add=True)

    plsc.subcore_barrier()
    pltpu.sync_copy(acc.at[pl.ds(s * RPS, RPS)],
                    out_hbm.at[pl.ds(c * NPAD + s * RPS, RPS)])


def _sc_deg(dp_deg, zeros128, ones128):
    return pl.kernel(
        _deg_body,
        out_type=jax.ShapeDtypeStruct((NC * NPAD, CH), jnp.float32),
        mesh=_mesh(),
        scratch_types=[
            pltpu.VMEM((BPS // NC, EB), jnp.int32),
            pltpu.VMEM((EB, CH), jnp.float32),
            pltpu.VMEM_SHARED((NPAD, CH), jnp.float32),
        ],
    )(dp_deg, zeros128, ones128)


SEG = 2
SROWS = BPS // SEG


def _prop_body(y_hbm, sp_hbm, dp_hbm, out_hbm, spv, dpv, buf, acc, sem0, sem1):
    c = lax.axis_index("c")
    s = lax.axis_index("s")
    for hc in range(FC // NC):
        fc = c * (FC // NC) + hc
        base = fc * NPAD
        pltpu.sync_copy(y_hbm.at[pl.ds(base + s * RPS, RPS)],
                        acc.at[pl.ds(s * RPS, RPS)])
        plsc.subcore_barrier()
        for seg in range(SEG):
            pltpu.sync_copy(sp_hbm.at[(fc * NS + s) * SEG + seg], spv)
            pltpu.sync_copy(dp_hbm.at[s * SEG + seg], dpv)
            pltpu.async_copy(y_hbm.at[spv.at[0]], buf.at[0], sem0)

            @pl.loop(0, SROWS // 2)
            def _(t):
                j = 2 * t
                pltpu.async_copy(y_hbm.at[spv.at[j + 1]], buf.at[1], sem1)
                pltpu.make_async_copy(y_hbm.at[spv.at[j]], buf.at[0], sem0).wait()
                pltpu.sync_copy(buf.at[0], acc.at[dpv.at[j]], add=True)

                @pl.when(j + 2 < SROWS)
                def _():
                    pltpu.async_copy(y_hbm.at[spv.at[j + 2]], buf.at[0], sem0)

                pltpu.make_async_copy(y_hbm.at[spv.at[j + 1]], buf.at[1], sem1).wait()
                pltpu.sync_copy(buf.at[1], acc.at[dpv.at[j + 1]], add=True)

        plsc.subcore_barrier()
        pltpu.sync_copy(acc.at[pl.ds(s * RPS, RPS)],
                        out_hbm.at[pl.ds(base + s * RPS, RPS)])


def _sc_prop(yflat, spseg, dpseg):
    return pl.kernel(
        _prop_body,
        out_type=jax.ShapeDtypeStruct((FC * NPAD, CH), jnp.float32),
        mesh=_mesh(),
        scratch_types=[
            pltpu.VMEM((SROWS, EB), jnp.int32),
            pltpu.VMEM((SROWS, EB), jnp.int32),
            pltpu.VMEM((2, EB, CH), jnp.float32),
            pltpu.VMEM_SHARED((NPAD, CH), jnp.float32),
            pltpu.SemaphoreType.DMA,
            pltpu.SemaphoreType.DMA,
        ],
    )(yflat, spseg, dpseg)


def _dinv_of(deg_ref):
    d = deg_ref[0] + deg_ref[1]
    return lax.rsqrt(d[:, 0:1] + 1.0)


def _enc_body(x_ref, W_ref, b_ref, Wg_ref, deg_ref, y_ref, dinv_ref):
    dinv = _dinv_of(deg_ref)
    dinv_ref[...] = jnp.broadcast_to(dinv, (BN, CH))
    h = jnp.dot(x_ref[...], W_ref[...], preferred_element_type=jnp.float32) + b_ref[0]
    y = jnp.dot(h, Wg_ref[...], preferred_element_type=jnp.float32) * dinv
    for f in range(FC):
        y_ref[f] = y[:, f * CH:(f + 1) * CH]


def _tc_enc(xpad, W_enc, b_enc, Wg0, degs):
    return pl.pallas_call(
        _enc_body,
        grid=(NB,),
        in_specs=[
            pl.BlockSpec((BN, F), lambda i: (i, 0)),
            pl.BlockSpec((F, H), lambda i: (0, 0)),
            pl.BlockSpec((1, H), lambda i: (0, 0)),
            pl.BlockSpec((H, H), lambda i: (0, 0)),
            pl.BlockSpec((NC, BN, CH), lambda i: (0, i, 0)),
        ],
        out_specs=[pl.BlockSpec((FC, BN, CH), lambda i: (0, i, 0)),
                   pl.BlockSpec((BN, CH), lambda i: (i, 0))],
        out_shape=[jax.ShapeDtypeStruct((FC, NPAD, CH), jnp.float32),
                   jax.ShapeDtypeStruct((NPAD, CH), jnp.float32)],
    )(xpad, W_enc, b_enc, Wg0, degs)


def _layer_body(prop_ref, dinv_ref, b_ref, Wg_ref, y_ref):
    dinv = dinv_ref[:, 0:1]
    hcat = jnp.concatenate([prop_ref[f] for f in range(FC)], axis=1)
    h = jnp.maximum(hcat * dinv + b_ref[0], 0.0)
    y = jnp.dot(h, Wg_ref[...], preferred_element_type=jnp.float32) * dinv
    for f in range(FC):
        y_ref[f] = y[:, f * CH:(f + 1) * CH]


def _tc_layer(prop, dinv, b_prev, Wg):
    return pl.pallas_call(
        _layer_body,
        grid=(NB,),
        in_specs=[
            pl.BlockSpec((FC, BN, CH), lambda i: (0, i, 0)),
            pl.BlockSpec((BN, CH), lambda i: (i, 0)),
            pl.BlockSpec((1, H), lambda i: (0, 0)),
            pl.BlockSpec((H, H), lambda i: (0, 0)),
        ],
        out_specs=pl.BlockSpec((FC, BN, CH), lambda i: (0, i, 0)),
        out_shape=jax.ShapeDtypeStruct((FC, NPAD, CH), jnp.float32),
    )(prop, dinv, b_prev, Wg)


def _head_body(prop_ref, dinv_ref, b_ref, A1_ref, a1_ref, A2_ref, a2_ref,
               V1_ref, v1_ref, V2_ref, v2_ref, batch_ref,
               out_ref, sums_h, sums_a, cnt, adv_s, val_s, am_s):
    i = pl.program_id(0)
    b = batch_ref[0, 0]
    oh = (b[:, None] == lax.broadcasted_iota(jnp.int32, (1, G), 1)
          ).astype(jnp.float32)

    @pl.when(i < NB)
    def _():
        dinv = dinv_ref[:, 0:1]
        hcat = jnp.concatenate([prop_ref[f] for f in range(FC)], axis=1)
        h = jnp.maximum(hcat * dinv + b_ref[0], 0.0)
        ap = jnp.maximum(
            jnp.dot(h, A1_ref[...], preferred_element_type=jnp.float32)
            + a1_ref[0], 0.0)
        adv = jnp.dot(ap, A2_ref[...], preferred_element_type=jnp.float32) + a2_ref[0]
        adv_s[pl.ds(i * BN, BN), :] = adv
        ph = lax.dot_general(oh, h, (((0,), (0,)), ((), ())),
                             preferred_element_type=jnp.float32)
        pa = lax.dot_general(oh, adv, (((0,), (0,)), ((), ())),
                             preferred_element_type=jnp.float32)
        pc = jnp.sum(oh, axis=0)[:, None]

        @pl.when(i == 0)
        def _():
            sums_h[...] = ph
            sums_a[...] = pa
            cnt[...] = pc

        @pl.when(i > 0)
        def _():
            sums_h[...] += ph
            sums_a[...] += pa
            cnt[...] += pc

        @pl.when(i == NB - 1)
        def _():
            c = jnp.maximum(cnt[...], 1.0)
            vx = sums_h[...] / c
            v = jnp.dot(
                jnp.maximum(
                    jnp.dot(vx, V1_ref[...], preferred_element_type=jnp.float32)
                    + v1_ref[0], 0.0),
                V2_ref[...], preferred_element_type=jnp.float32) + v2_ref[0]
            val_s[...] = v
            am_s[...] = sums_a[...] / c

    @pl.when(i >= NB)
    def _():
        j = i - NB
        delta = val_s[...] - am_s[...]
        out_ref[...] = adv_s[pl.ds(j * BN, BN), :] + jnp.dot(
            oh, delta, preferred_element_type=jnp.float32)


def _tc_head(prop, dinv, b3, A1, a1, A2, a2, V1, v1, V2, v2, batch3):
    hold = lambda i: (jnp.minimum(i, NB - 1), 0)
    hold3 = lambda i: (0, jnp.minimum(i, NB - 1), 0)
    return pl.pallas_call(
        _head_body,
        grid=(2 * NB,),
        in_specs=[
            pl.BlockSpec((FC, BN, CH), hold3),
            pl.BlockSpec((BN, CH), hold),
            pl.BlockSpec((1, H), lambda i: (0, 0)),
            pl.BlockSpec((H, H), lambda i: (0, 0)),
            pl.BlockSpec((1, H), lambda i: (0, 0)),
            pl.BlockSpec((H, 1), lambda i: (0, 0)),
            pl.BlockSpec((1, 1), lambda i: (0, 0)),
            pl.BlockSpec((H, H), lambda i: (0, 0)),
            pl.BlockSpec((1, H), lambda i: (0, 0)),
            pl.BlockSpec((H, 1), lambda i: (0, 0)),
            pl.BlockSpec((1, 1), lambda i: (0, 0)),
            pl.BlockSpec((1, 1, BN),
                         lambda i: (jnp.where(i < NB, i, i - NB), 0, 0)),
        ],
        out_specs=pl.BlockSpec((BN, 1), lambda i: (jnp.maximum(i - NB, 0), 0)),
        out_shape=jax.ShapeDtypeStruct((NPAD, 1), jnp.float32),
        scratch_shapes=[
            pltpu.VMEM((G, H), jnp.float32),
            pltpu.VMEM((G, 1), jnp.float32),
            pltpu.VMEM((G, 1), jnp.float32),
            pltpu.VMEM((NPAD, 1), jnp.float32),
            pltpu.VMEM((G, 1), jnp.float32),
            pltpu.VMEM((G, 1), jnp.float32),
        ],
    )(prop, dinv, b3, A1, a1, A2, a2, V1, v1, V2, v2, batch3)


def kernel(x, edge_index, batch, W_enc, b_enc, Wg0, bg0, Wg1, bg1, Wg2, bg2,
           Wg3, bg3, A1, a1, A2, a2, V1, v1, V2, v2):
    src, dst = edge_index[0], edge_index[1]
    pad_idx = (N + jnp.arange(EPAD - E, dtype=jnp.int32) % (NPAD - N))
    sp = jnp.concatenate([src, pad_idx])
    dp = jnp.concatenate([dst, pad_idx])
    spseg = (sp[None, :]
             + (jnp.arange(FC, dtype=jnp.int32) * NPAD)[:, None]
             ).reshape(FC * NS * SEG, SROWS, EB)
    dpseg = dp.reshape(NS * SEG, SROWS, EB)
    xpad = jnp.concatenate([x, jnp.zeros((NPAD - N, F), x.dtype)])
    batch3 = jnp.concatenate(
        [batch, jnp.full((NPAD - N,), G, jnp.int32)]).reshape(NB, 1, BN)
    zeros128 = jnp.zeros((NPAD, CH), jnp.float32)
    ones128 = jnp.ones((EB, CH), jnp.float32)

    dp_deg = dp.reshape(NC * NS, BPS // NC, EB)
    degs = _sc_deg(dp_deg, zeros128, ones128).reshape(NC, NPAD, CH)
    y, dinv = _tc_enc(xpad, W_enc, b_enc.reshape(1, H), Wg0, degs)
    for Wg, b_prev in ((Wg1, bg0), (Wg2, bg1), (Wg3, bg2)):
        prop = _sc_prop(y.reshape(FC * NPAD, CH), spseg, dpseg).reshape(FC, NPAD, CH)
        y = _tc_layer(prop, dinv, b_prev.reshape(1, H), Wg)
    prop = _sc_prop(y.reshape(FC * NPAD, CH), spseg, dpseg).reshape(FC, NPAD, CH)
    out = _tc_head(prop, dinv, bg3.reshape(1, H), A1,
                   a1.reshape(1, H), A2, a2.reshape(1, 1),
                   V1, v1.reshape(1, H), V2, v2.reshape(1, 1), batch3)
    return out[:N]
